# baseline (speedup 1.0000x reference)
import functools

import jax
import jax.numpy as jnp
from jax import lax
from jax.experimental import pallas as pl
from jax.experimental.pallas import tpu as pltpu
from jax.experimental.pallas import tpu_sc as plsc

BATCH = 4096
FIELDS = 26
EMB_DIM = 32

NC = 2
NS = 16
NW = NC * NS
BATCH_PER_W = BATCH // NW
CB = 8
NCHUNK = BATCH_PER_W // CB


def _make_lookup():
  mesh = plsc.VectorSubcoreMesh(core_axis_name="c", subcore_axis_name="s")

  @functools.partial(
      pl.kernel,
      mesh=mesh,
      out_type=jax.ShapeDtypeStruct((BATCH, FIELDS, EMB_DIM), jnp.float32),
      compiler_params=pltpu.CompilerParams(
          disable_bounds_checks=True,
          disable_semaphore_checks=True,
          skip_device_barrier=True,
      ),
      scratch_types=[
          pltpu.VMEM((BATCH_PER_W * FIELDS,), jnp.int32),
          [pltpu.VMEM((CB, FIELDS, EMB_DIM), jnp.float32) for _ in range(2)],
          [pltpu.SemaphoreType.DMA for _ in range(2)],
          [pltpu.SemaphoreType.DMA for _ in range(2)],
      ],
  )
  def lookup(idx_hbm, table_hbm, out_hbm, idx_v, bufs, insems, outsems):
    wid = lax.axis_index("s") * NC + lax.axis_index("c")
    base = wid * BATCH_PER_W
    pltpu.sync_copy(
        idx_hbm.at[pl.ds(base * FIELDS, BATCH_PER_W * FIELDS)], idx_v)

    def issue_chunk(c, buf, insem):
      def body(g, _):
        v = idx_v[pl.ds(c * (CB * FIELDS) + g * 16, 16)]
        for j in range(16):
          p = g * 16 + j
          bb = p // FIELDS
          f = p - bb * FIELDS
          pltpu.async_copy(
              table_hbm.at[pl.ds(v[j] * EMB_DIM, EMB_DIM)],
              buf.at[bb, f], insem)
        return 0

      lax.fori_loop(0, CB * FIELDS // 16, body, 0)

    def drain_chunk(buf, insem):
      def body(r, _):
        pltpu.make_async_copy(
            table_hbm.at[pl.ds(0, EMB_DIM)],
            buf.at[0, 0], insem).wait()
        return 0

      lax.fori_loop(0, CB * FIELDS, body, 0, unroll=8)

    for c in range(NCHUNK + 1):
      if c < NCHUNK:
        p = c % 2
        if c >= 2:
          pltpu.make_async_copy(
              bufs[p], out_hbm.at[pl.ds(base, CB)], outsems[p]).wait()
        issue_chunk(c, bufs[p], insems[p])
      if c >= 1:
        q = (c - 1) % 2
        drain_chunk(bufs[q], insems[q])
        pltpu.async_copy(
            bufs[q], out_hbm.at[pl.ds(base + (c - 1) * CB, CB)], outsems[q])
    for c in (NCHUNK - 2, NCHUNK - 1):
      p = c % 2
      pltpu.make_async_copy(
          bufs[p], out_hbm.at[pl.ds(base, CB)], outsems[p]).wait()

  return lookup


_lookup = _make_lookup()


@jax.jit
def kernel(x, weight):
  idx = x.reshape(BATCH * FIELDS).astype(jnp.int32)
  return _lookup(idx, weight.reshape(1000000 * EMB_DIM))


# speedup vs baseline: 1.5100x; 1.5100x over previous
import functools

import jax
import jax.numpy as jnp
from jax import lax
from jax.experimental import pallas as pl
from jax.experimental.pallas import tpu as pltpu
from jax.experimental.pallas import tpu_sc as plsc

BATCH = 4096
FIELDS = 26
EMB_DIM = 32

NC = 2
NS = 16
NW = NC * NS
BATCH_PER_W = BATCH // NW
CB = 8
NCHUNK = BATCH_PER_W // CB


def _make_lookup():
  mesh = plsc.VectorSubcoreMesh(core_axis_name="c", subcore_axis_name="s")

  @functools.partial(
      pl.kernel,
      mesh=mesh,
      out_type=jax.ShapeDtypeStruct((BATCH, FIELDS, EMB_DIM), jnp.float32),
      scratch_types=[
          pltpu.VMEM((BATCH_PER_W * FIELDS,), jnp.int32),
          [pltpu.VMEM((CB, FIELDS, EMB_DIM), jnp.float32) for _ in range(2)],
          [pltpu.SemaphoreType.DMA for _ in range(2)],
          [pltpu.SemaphoreType.DMA for _ in range(2)],
      ],
  )
  def lookup(idx_hbm, table_hbm, out_hbm, idx_v, bufs, insems, outsems):
    wid = lax.axis_index("s") * NC + lax.axis_index("c")
    base = wid * BATCH_PER_W
    pltpu.sync_copy(
        idx_hbm.at[pl.ds(base * FIELDS, BATCH_PER_W * FIELDS)], idx_v)

    def issue_chunk(c, buf, insem):
      def body(g, _):
        v = idx_v[pl.ds(c * (CB * FIELDS) + g * 16, 16)]
        for j in range(16):
          p = g * 16 + j
          bb = p // FIELDS
          f = p - bb * FIELDS
          pltpu.async_copy(
              table_hbm.at[pl.ds(v[j], 1)],
              buf.at[bb, pl.ds(f, 1)], insem)
        return 0

      lax.fori_loop(0, CB * FIELDS // 16, body, 0)

    def drain_chunk(buf, insem):
      def body(r, _):
        pltpu.make_async_copy(
            table_hbm.at[pl.ds(0, 1)],
            buf.at[0, pl.ds(0, 1)], insem).wait()
        return 0

      lax.fori_loop(0, CB * FIELDS, body, 0, unroll=8)

    for c in range(NCHUNK + 1):
      if c < NCHUNK:
        p = c % 2
        if c >= 2:
          pltpu.make_async_copy(
              bufs[p], out_hbm.at[pl.ds(base, CB)], outsems[p]).wait()
        issue_chunk(c, bufs[p], insems[p])
      if c >= 1:
        q = (c - 1) % 2
        drain_chunk(bufs[q], insems[q])
        pltpu.async_copy(
            bufs[q], out_hbm.at[pl.ds(base + (c - 1) * CB, CB)], outsems[q])
    for c in (NCHUNK - 2, NCHUNK - 1):
      p = c % 2
      pltpu.make_async_copy(
          bufs[p], out_hbm.at[pl.ds(base, CB)], outsems[p]).wait()

  return lookup


_lookup = _make_lookup()


@jax.jit
def kernel(x, weight):
  idx = x.reshape(BATCH * FIELDS).astype(jnp.int32)
  return _lookup(idx, weight)


# speedup vs baseline: 2.0626x; 1.3659x over previous
import functools

import jax
import jax.numpy as jnp
from jax import lax
from jax.experimental import pallas as pl
from jax.experimental.pallas import tpu as pltpu
from jax.experimental.pallas import tpu_sc as plsc

BATCH = 4096
FIELDS = 26
EMB_DIM = 32

NC = 2
NS = 16
NW = NC * NS
BATCH_PER_W = BATCH // NW
CB = 8
NCHUNK = BATCH_PER_W // CB


def _make_lookup():
  mesh = plsc.VectorSubcoreMesh(core_axis_name="c", subcore_axis_name="s")

  @functools.partial(
      pl.kernel,
      mesh=mesh,
      out_type=jax.ShapeDtypeStruct((BATCH, FIELDS, EMB_DIM), jnp.float32),
      scratch_types=[
          pltpu.VMEM((BATCH_PER_W * FIELDS,), jnp.int32),
          [pltpu.VMEM((CB, FIELDS, EMB_DIM), jnp.float32) for _ in range(2)],
          [pltpu.SemaphoreType.DMA for _ in range(2)],
          [pltpu.SemaphoreType.DMA for _ in range(2)],
      ],
  )
  def lookup(idx_hbm, table_hbm, out_hbm, idx_v, bufs, insems, outsems):
    wid = lax.axis_index("s") * NC + lax.axis_index("c")
    base = wid * BATCH_PER_W
    pltpu.sync_copy(
        idx_hbm.at[pl.ds(base * FIELDS, BATCH_PER_W * FIELDS)], idx_v)

    def issue_chunk(c, buf, insem):
      def body(g, _):
        v = idx_v[pl.ds(c * (CB * FIELDS) + g * 16, 16)]
        for j in range(16):
          p = g * 16 + j
          bb = p // FIELDS
          f = p - bb * FIELDS
          i = v[j]
          pltpu.async_copy(
              table_hbm.at[i // 8, i % 8],
              buf.at[bb, f], insem)
        return 0

      lax.fori_loop(0, CB * FIELDS // 16, body, 0)

    def drain_chunk(buf, insem):
      def body(r, _):
        pltpu.make_async_copy(
            table_hbm.at[0, 0],
            buf.at[0, 0], insem).wait()
        return 0

      lax.fori_loop(0, CB * FIELDS, body, 0, unroll=8)

    for c in range(NCHUNK + 1):
      if c < NCHUNK:
        p = c % 2
        if c >= 2:
          pltpu.make_async_copy(
              bufs[p], out_hbm.at[pl.ds(base, CB)], outsems[p]).wait()
        issue_chunk(c, bufs[p], insems[p])
      if c >= 1:
        q = (c - 1) % 2
        drain_chunk(bufs[q], insems[q])
        pltpu.async_copy(
            bufs[q], out_hbm.at[pl.ds(base + (c - 1) * CB, CB)], outsems[q])
    for c in (NCHUNK - 2, NCHUNK - 1):
      p = c % 2
      pltpu.make_async_copy(
          bufs[p], out_hbm.at[pl.ds(base, CB)], outsems[p]).wait()

  return lookup


_lookup = _make_lookup()


@jax.jit
def kernel(x, weight):
  idx = x.reshape(BATCH * FIELDS).astype(jnp.int32)
  return _lookup(idx, weight.reshape(125000, 8, EMB_DIM))


# speedup vs baseline: 2.2001x; 1.0667x over previous
import functools

import jax
import jax.numpy as jnp
from jax import lax
from jax.experimental import pallas as pl
from jax.experimental.pallas import tpu as pltpu
from jax.experimental.pallas import tpu_sc as plsc

BATCH = 4096
FIELDS = 26
EMB_DIM = 32

NC = 2
NS = 16
NW = NC * NS
BATCH_PER_W = BATCH // NW
CB = 8
NCHUNK = BATCH_PER_W // CB


def _make_lookup():
  mesh = plsc.VectorSubcoreMesh(core_axis_name="c", subcore_axis_name="s")

  @functools.partial(
      pl.kernel,
      mesh=mesh,
      out_type=jax.ShapeDtypeStruct((BATCH, FIELDS, EMB_DIM), jnp.float32),
      scratch_types=[
          pltpu.VMEM((BATCH_PER_W * FIELDS,), jnp.int32),
          [pltpu.VMEM((CB, FIELDS, EMB_DIM), jnp.float32) for _ in range(2)],
          [pltpu.SemaphoreType.DMA for _ in range(2)],
          [pltpu.SemaphoreType.DMA for _ in range(2)],
      ],
  )
  def lookup(idx_hbm, table_hbm, out_hbm, idx_v, bufs, insems, outsems):
    wid = lax.axis_index("s") * NC + lax.axis_index("c")
    base = wid * BATCH_PER_W
    pltpu.sync_copy(
        idx_hbm.at[pl.ds(base * FIELDS, BATCH_PER_W * FIELDS)], idx_v)

    def issue_chunk(c, buf, insem):
      def body(g, _):
        v = idx_v[pl.ds(c * (CB * FIELDS) + g * 16, 16)]
        for j in range(16):
          p = g * 16 + j
          bb = p // FIELDS
          f = p - bb * FIELDS
          i = v[j]
          pltpu.async_copy(
              table_hbm.at[lax.shift_right_logical(i, 3), i & 7],
              buf.at[bb, f], insem)
        return 0

      lax.fori_loop(0, CB * FIELDS // 16, body, 0)

    def drain_chunk(buf, insem):
      def body(r, _):
        pltpu.make_async_copy(
            table_hbm.at[0, 0],
            buf.at[0, 0], insem).wait()
        return 0

      lax.fori_loop(0, CB * FIELDS, body, 0, unroll=8)

    for c in range(NCHUNK + 1):
      if c < NCHUNK:
        p = c % 2
        if c >= 2:
          pltpu.make_async_copy(
              bufs[p], out_hbm.at[pl.ds(base, CB)], outsems[p]).wait()
        issue_chunk(c, bufs[p], insems[p])
      if c >= 1:
        q = (c - 1) % 2
        drain_chunk(bufs[q], insems[q])
        pltpu.async_copy(
            bufs[q], out_hbm.at[pl.ds(base + (c - 1) * CB, CB)], outsems[q])
    for c in (NCHUNK - 2, NCHUNK - 1):
      p = c % 2
      pltpu.make_async_copy(
          bufs[p], out_hbm.at[pl.ds(base, CB)], outsems[p]).wait()

  return lookup


_lookup = _make_lookup()


@jax.jit
def kernel(x, weight):
  idx = x.reshape(BATCH * FIELDS).astype(jnp.int32)
  return _lookup(idx, weight.reshape(125000, 8, EMB_DIM))
